# baseline (speedup 1.0000x reference)
import functools

import jax
import jax.numpy as jnp
from jax import lax
from jax.experimental import pallas as pl
from jax.experimental.pallas import tpu as pltpu
from jax.experimental.pallas import tpu_sc as plsc

_ROWS = 64
_LANES = 16


def _gather_kernel(cam_ref, id_ref, w_ref, b_ref, ws_ref, bs_ref):
    bidx = pl.program_id(0)
    rem = id_ref[bidx] % 8
    sel = jax.lax.broadcasted_iota(jnp.int32, (8, 3), 0) == rem
    ws_ref[0] = jnp.sum(jnp.where(sel, w_ref[0], 0.0), axis=0, keepdims=True)
    bs_ref[0] = jnp.sum(jnp.where(sel, b_ref[0], 0.0), axis=0, keepdims=True)


def _gather_params(camindex, idindex, w, b):
    B = camindex.shape[0]
    return pl.pallas_call(
        _gather_kernel,
        grid_spec=pltpu.PrefetchScalarGridSpec(
            num_scalar_prefetch=2,
            grid=(B,),
            in_specs=[
                pl.BlockSpec((1, 8, 3), lambda bi, cam, idx: (cam[bi], idx[bi] // 8, 0)),
                pl.BlockSpec((1, 8, 3), lambda bi, cam, idx: (cam[bi], idx[bi] // 8, 0)),
            ],
            out_specs=[
                pl.BlockSpec((1, 1, 3), lambda bi, cam, idx: (bi, 0, 0)),
                pl.BlockSpec((1, 1, 3), lambda bi, cam, idx: (bi, 0, 0)),
            ],
        ),
        out_shape=[jax.ShapeDtypeStruct((B, 1, 3), jnp.float32)] * 2,
    )(camindex, idindex, w, b)


def _make_sc_affine(B, C, H, W):
    info = plsc.get_sparse_core_info()
    NC, NS = info.num_cores, info.num_subcores
    NW = NC * NS
    n_planes = B * C
    ppw = -(-n_planes // NW)
    n_chunks = H // _ROWS
    vecs_per_chunk = _ROWS * W // _LANES
    lane_chunks = W // _LANES

    mesh = plsc.VectorSubcoreMesh(core_axis_name="c", subcore_axis_name="s")

    @functools.partial(
        pl.kernel,
        mesh=mesh,
        out_type=jax.ShapeDtypeStruct((B, C, H, W), jnp.float32),
        scratch_types=[
            pltpu.VMEM((n_planes, 128), jnp.float32),
            pltpu.VMEM((n_planes, 128), jnp.float32),
            pltpu.VMEM((_ROWS, W), jnp.float32),
        ],
        compiler_params=pltpu.CompilerParams(
            use_tc_tiling_on_sc=True, needs_layout_passes=False
        ),
    )
    def sc_affine(ws_hbm, bs_hbm, img_hbm, out_hbm, wsv, bsv, buf):
        wid = lax.axis_index("s") * NC + lax.axis_index("c")
        pltpu.sync_copy(ws_hbm, wsv)
        pltpu.sync_copy(bs_hbm, bsv)
        for j in range(ppw):
            p = wid * ppw + j

            @pl.when(p < n_planes)
            def _():
                s = p // C
                c = p % C
                wv = wsv[p, pl.ds(0, _LANES)]
                bv = bsv[p, pl.ds(0, _LANES)]
                for k in range(n_chunks):
                    pltpu.sync_copy(img_hbm.at[s, c, pl.ds(k * _ROWS, _ROWS)], buf)

                    def body(i, carry):
                        r = i // lane_chunks
                        l = (i % lane_chunks) * _LANES
                        buf[r, pl.ds(l, _LANES)] = buf[r, pl.ds(l, _LANES)] * wv + bv
                        return carry

                    lax.fori_loop(0, vecs_per_chunk, body, 0)
                    pltpu.sync_copy(buf, out_hbm.at[s, c, pl.ds(k * _ROWS, _ROWS)])

    return sc_affine


def kernel(image, camindex, idindex, w, b):
    B, C, H, W = image.shape
    ws, bs = _gather_params(camindex, idindex, w, b)
    ws16 = jnp.broadcast_to(ws.reshape(B * C, 1), (B * C, 128))
    bs16 = jnp.broadcast_to(bs.reshape(B * C, 1), (B * C, 128))
    sc_affine = _make_sc_affine(B, C, H, W)
    return sc_affine(ws16, bs16, image)


# speedup vs baseline: 2.1554x; 2.1554x over previous
import functools

import jax
import jax.numpy as jnp
from jax import lax
from jax.experimental import pallas as pl
from jax.experimental.pallas import tpu as pltpu
from jax.experimental.pallas import tpu_sc as plsc

_ROWS = 64
_LANES = 16


def _gather_kernel(cam_ref, id_ref, wt_ref, bt_ref, ws_ref, bs_ref):
    bidx = pl.program_id(0)
    rrem = cam_ref[bidx] % 8
    crem = id_ref[bidx] % 128
    sel = (jax.lax.broadcasted_iota(jnp.int32, (8, 128), 0) == rrem) & (
        jax.lax.broadcasted_iota(jnp.int32, (8, 128), 1) == crem
    )
    for c in range(3):
        ws_ref[0, pl.ds(c, 1), :] = jnp.full(
            (1, 128), jnp.sum(jnp.where(sel, wt_ref[c], 0.0))
        )
        bs_ref[0, pl.ds(c, 1), :] = jnp.full(
            (1, 128), jnp.sum(jnp.where(sel, bt_ref[c], 0.0))
        )


def _gather_params(camindex, idindex, w, b):
    B = camindex.shape[0]
    wt = jnp.transpose(w, (2, 0, 1))
    bt = jnp.transpose(b, (2, 0, 1))
    return pl.pallas_call(
        _gather_kernel,
        grid_spec=pltpu.PrefetchScalarGridSpec(
            num_scalar_prefetch=2,
            grid=(B,),
            in_specs=[
                pl.BlockSpec(
                    (3, 8, 128), lambda bi, cam, idx: (0, cam[bi] // 8, idx[bi] // 128)
                ),
                pl.BlockSpec(
                    (3, 8, 128), lambda bi, cam, idx: (0, cam[bi] // 8, idx[bi] // 128)
                ),
            ],
            out_specs=[
                pl.BlockSpec((1, 3, 128), lambda bi, cam, idx: (bi, 0, 0)),
                pl.BlockSpec((1, 3, 128), lambda bi, cam, idx: (bi, 0, 0)),
            ],
        ),
        out_shape=[jax.ShapeDtypeStruct((B, 3, 128), jnp.float32)] * 2,
    )(camindex, idindex, wt, bt)


def _make_sc_affine(B, C, H, W):
    info = plsc.get_sparse_core_info()
    NC, NS = info.num_cores, info.num_subcores
    NW = NC * NS
    n_planes = B * C
    ppw = -(-n_planes // NW)
    n_chunks = H // _ROWS
    vecs_per_chunk = _ROWS * W // _LANES
    lane_chunks = W // _LANES

    mesh = plsc.VectorSubcoreMesh(core_axis_name="c", subcore_axis_name="s")

    @functools.partial(
        pl.kernel,
        mesh=mesh,
        out_type=jax.ShapeDtypeStruct((B, C, H, W), jnp.float32),
        scratch_types=[
            pltpu.VMEM((n_planes, 128), jnp.float32),
            pltpu.VMEM((n_planes, 128), jnp.float32),
            pltpu.VMEM((_ROWS, W), jnp.float32),
        ],
        compiler_params=pltpu.CompilerParams(
            use_tc_tiling_on_sc=True, needs_layout_passes=False
        ),
    )
    def sc_affine(ws_hbm, bs_hbm, img_hbm, out_hbm, wsv, bsv, buf):
        wid = lax.axis_index("s") * NC + lax.axis_index("c")
        pltpu.sync_copy(ws_hbm, wsv)
        pltpu.sync_copy(bs_hbm, bsv)
        for j in range(ppw):
            p = wid * ppw + j

            @pl.when(p < n_planes)
            def _():
                s = p // C
                c = p % C
                wv = wsv[p, pl.ds(0, _LANES)]
                bv = bsv[p, pl.ds(0, _LANES)]
                for k in range(n_chunks):
                    pltpu.sync_copy(img_hbm.at[s, c, pl.ds(k * _ROWS, _ROWS)], buf)

                    def body(i, carry):
                        r = i // lane_chunks
                        l = (i % lane_chunks) * _LANES
                        buf[r, pl.ds(l, _LANES)] = buf[r, pl.ds(l, _LANES)] * wv + bv
                        return carry

                    lax.fori_loop(0, vecs_per_chunk, body, 0)
                    pltpu.sync_copy(buf, out_hbm.at[s, c, pl.ds(k * _ROWS, _ROWS)])

    return sc_affine


def kernel(image, camindex, idindex, w, b):
    B, C, H, W = image.shape
    ws, bs = _gather_params(camindex, idindex, w, b)
    sc_affine = _make_sc_affine(B, C, H, W)
    return sc_affine(ws.reshape(B * C, 128), bs.reshape(B * C, 128), image)


# speedup vs baseline: 8.0805x; 3.7490x over previous
import functools

import jax
import jax.numpy as jnp
from jax import lax
from jax.experimental import pallas as pl
from jax.experimental.pallas import tpu as pltpu
from jax.experimental.pallas import tpu_sc as plsc

_ROWS = 64
_LANES = 16


def _gather_kernel(cam_ref, id_ref, wt_ref, bt_ref, ws_ref, bs_ref):
    bidx = pl.program_id(0)
    rrem = cam_ref[bidx] % 8
    crem = id_ref[bidx] % 128
    sel = (jax.lax.broadcasted_iota(jnp.int32, (8, 128), 0) == rrem) & (
        jax.lax.broadcasted_iota(jnp.int32, (8, 128), 1) == crem
    )
    for c in range(3):
        ws_ref[0, pl.ds(c, 1), :] = jnp.full(
            (1, 128), jnp.sum(jnp.where(sel, wt_ref[c], 0.0))
        )
        bs_ref[0, pl.ds(c, 1), :] = jnp.full(
            (1, 128), jnp.sum(jnp.where(sel, bt_ref[c], 0.0))
        )


def _gather_params(camindex, idindex, w, b):
    B = camindex.shape[0]
    wt = jnp.transpose(w, (2, 0, 1))
    bt = jnp.transpose(b, (2, 0, 1))
    return pl.pallas_call(
        _gather_kernel,
        grid_spec=pltpu.PrefetchScalarGridSpec(
            num_scalar_prefetch=2,
            grid=(B,),
            in_specs=[
                pl.BlockSpec(
                    (3, 8, 128), lambda bi, cam, idx: (0, cam[bi] // 8, idx[bi] // 128)
                ),
                pl.BlockSpec(
                    (3, 8, 128), lambda bi, cam, idx: (0, cam[bi] // 8, idx[bi] // 128)
                ),
            ],
            out_specs=[
                pl.BlockSpec((1, 3, 128), lambda bi, cam, idx: (bi, 0, 0)),
                pl.BlockSpec((1, 3, 128), lambda bi, cam, idx: (bi, 0, 0)),
            ],
        ),
        out_shape=[jax.ShapeDtypeStruct((B, 3, 128), jnp.float32)] * 2,
    )(camindex, idindex, wt, bt)


def _make_sc_affine(B, C, H, W):
    info = plsc.get_sparse_core_info()
    NC, NS = info.num_cores, info.num_subcores
    NW = NC * NS
    n_planes = B * C
    assert n_planes % NW == 0 and n_planes // NW == C
    n_chunks = H // _ROWS
    lane_chunks = W // _LANES
    n_total = C * n_chunks

    mesh = plsc.VectorSubcoreMesh(core_axis_name="c", subcore_axis_name="s")

    @functools.partial(
        pl.kernel,
        mesh=mesh,
        out_type=jax.ShapeDtypeStruct((B, C, H, W), jnp.float32),
        scratch_types=[
            pltpu.VMEM((n_planes, 128), jnp.float32),
            pltpu.VMEM((n_planes, 128), jnp.float32),
            pltpu.VMEM((_ROWS, W), jnp.float32),
            pltpu.VMEM((_ROWS, W), jnp.float32),
            pltpu.SemaphoreType.DMA,
            pltpu.SemaphoreType.DMA,
            pltpu.SemaphoreType.DMA,
            pltpu.SemaphoreType.DMA,
        ],
        compiler_params=pltpu.CompilerParams(
            use_tc_tiling_on_sc=True, needs_layout_passes=False
        ),
    )
    def sc_affine(ws_hbm, bs_hbm, img_hbm, out_hbm, wsv, bsv, bufA, bufB, inA, inB, outA, outB):
        wid = lax.axis_index("s") * NC + lax.axis_index("c")
        s = wid
        pltpu.sync_copy(ws_hbm, wsv)
        pltpu.sync_copy(bs_hbm, bsv)
        bufs = (bufA, bufB)
        insems = (inA, inB)
        outsems = (outA, outB)

        def coords(k):
            return k // n_chunks, (k % n_chunks) * _ROWS

        def in_copy(k, b):
            c, r0 = coords(k)
            return pltpu.make_async_copy(
                img_hbm.at[s, c, pl.ds(r0, _ROWS)], bufs[b], insems[b]
            )

        def out_copy(k, b):
            c, r0 = coords(k)
            return pltpu.make_async_copy(
                bufs[b], out_hbm.at[s, c, pl.ds(r0, _ROWS)], outsems[b]
            )

        def compute(buf, wv, bv):
            def rowbody(r, carry):
                for t in range(lane_chunks):
                    sl = pl.ds(t * _LANES, _LANES)
                    buf[r, sl] = buf[r, sl] * wv + bv
                return carry

            lax.fori_loop(0, _ROWS, rowbody, 0)

        in_copy(0, 0).start()
        for k in range(n_total):
            b = k % 2
            nb = (k + 1) % 2
            if k + 1 < n_total:
                if k >= 1:
                    out_copy(k - 1, nb).wait()
                in_copy(k + 1, nb).start()
            in_copy(k, b).wait()
            c, _ = coords(k)
            p = wid * C + c
            compute(bufs[b], wsv[p, pl.ds(0, _LANES)], bsv[p, pl.ds(0, _LANES)])
            out_copy(k, b).start()
        out_copy(n_total - 2, (n_total - 2) % 2).wait()
        out_copy(n_total - 1, (n_total - 1) % 2).wait()

    return sc_affine


def kernel(image, camindex, idindex, w, b):
    B, C, H, W = image.shape
    ws, bs = _gather_params(camindex, idindex, w, b)
    sc_affine = _make_sc_affine(B, C, H, W)
    return sc_affine(ws.reshape(B * C, 128), bs.reshape(B * C, 128), image)


# speedup vs baseline: 9.5248x; 1.1788x over previous
import functools

import jax
import jax.numpy as jnp
from jax import lax
from jax.experimental import pallas as pl
from jax.experimental.pallas import tpu as pltpu
from jax.experimental.pallas import tpu_sc as plsc

_ROWS = 64
_LANES = 16
_NBUF = 3


def _make_gather(B, C):
    def gather_kernel(cam_ref, id_ref, *refs):
        w_blocks = refs[:B]
        b_blocks = refs[B : 2 * B]
        ws_ref, bs_ref = refs[2 * B], refs[2 * B + 1]
        for s in range(B):
            rrem = cam_ref[s] % 8
            crem = id_ref[s] % 128
            sel = (jax.lax.broadcasted_iota(jnp.int32, (8, 128), 0) == rrem) & (
                jax.lax.broadcasted_iota(jnp.int32, (8, 128), 1) == crem
            )
            for c in range(C):
                ws_ref[pl.ds(s * C + c, 1), :] = jnp.full(
                    (1, 128), jnp.sum(jnp.where(sel, w_blocks[s][c], 0.0))
                )
                bs_ref[pl.ds(s * C + c, 1), :] = jnp.full(
                    (1, 128), jnp.sum(jnp.where(sel, b_blocks[s][c], 0.0))
                )

    return gather_kernel


def _gather_params(camindex, idindex, w, b):
    B = camindex.shape[0]
    C = w.shape[-1]
    wt = jnp.transpose(w, (2, 0, 1))
    bt = jnp.transpose(b, (2, 0, 1))

    def make_spec(s):
        return pl.BlockSpec(
            (C, 8, 128), lambda bi, cam, idx, s=s: (0, cam[s] // 8, idx[s] // 128)
        )

    return pl.pallas_call(
        _make_gather(B, C),
        grid_spec=pltpu.PrefetchScalarGridSpec(
            num_scalar_prefetch=2,
            grid=(1,),
            in_specs=[make_spec(s) for s in range(B)] * 2,
            out_specs=[
                pl.BlockSpec((B * C, 128), lambda bi, cam, idx: (0, 0)),
                pl.BlockSpec((B * C, 128), lambda bi, cam, idx: (0, 0)),
            ],
        ),
        out_shape=[jax.ShapeDtypeStruct((B * C, 128), jnp.float32)] * 2,
    )(camindex, idindex, *([wt] * B), *([bt] * B))


def _make_sc_affine(B, C, H, W):
    info = plsc.get_sparse_core_info()
    NC, NS = info.num_cores, info.num_subcores
    NW = NC * NS
    n_planes = B * C
    assert n_planes % NW == 0 and n_planes // NW == C
    n_chunks = H // _ROWS
    lane_chunks = W // _LANES
    n_total = C * n_chunks

    mesh = plsc.VectorSubcoreMesh(core_axis_name="c", subcore_axis_name="s")

    @functools.partial(
        pl.kernel,
        mesh=mesh,
        out_type=jax.ShapeDtypeStruct((B, C, H, W), jnp.float32),
        scratch_types=[
            pltpu.VMEM((n_planes, 128), jnp.float32),
            pltpu.VMEM((n_planes, 128), jnp.float32),
        ]
        + [pltpu.VMEM((_ROWS, W), jnp.float32)] * _NBUF
        + [pltpu.SemaphoreType.DMA] * (2 * _NBUF),
        compiler_params=pltpu.CompilerParams(
            use_tc_tiling_on_sc=True, needs_layout_passes=False
        ),
    )
    def sc_affine(ws_hbm, bs_hbm, img_hbm, out_hbm, wsv, bsv, *bufsem):
        bufs = bufsem[:_NBUF]
        insems = bufsem[_NBUF : 2 * _NBUF]
        outsems = bufsem[2 * _NBUF :]
        wid = lax.axis_index("s") * NC + lax.axis_index("c")
        s = wid
        pltpu.sync_copy(ws_hbm, wsv)
        pltpu.sync_copy(bs_hbm, bsv)

        def coords(k):
            return k // n_chunks, (k % n_chunks) * _ROWS

        def in_copy(k, b):
            c, r0 = coords(k)
            return pltpu.make_async_copy(
                img_hbm.at[s, c, pl.ds(r0, _ROWS)], bufs[b], insems[b]
            )

        def out_copy(k, b):
            c, r0 = coords(k)
            return pltpu.make_async_copy(
                bufs[b], out_hbm.at[s, c, pl.ds(r0, _ROWS)], outsems[b]
            )

        def compute(buf, wv, bv):
            def rowbody(r, carry):
                for t in range(lane_chunks):
                    sl = pl.ds(t * _LANES, _LANES)
                    buf[r, sl] = buf[r, sl] * wv + bv
                return carry

            lax.fori_loop(0, _ROWS, rowbody, 0)

        for k in range(_NBUF - 1):
            in_copy(k, k % _NBUF).start()
        for k in range(n_total):
            b = k % _NBUF
            in_copy(k, b).wait()
            c, _ = coords(k)
            p = wid * C + c
            compute(bufs[b], wsv[p, pl.ds(0, _LANES)], bsv[p, pl.ds(0, _LANES)])
            out_copy(k, b).start()
            nxt = k + _NBUF - 1
            if nxt < n_total:
                nb = nxt % _NBUF
                if nxt >= _NBUF:
                    out_copy(nxt - _NBUF, nb).wait()
                in_copy(nxt, nb).start()
        for k in range(max(0, n_total - _NBUF), n_total):
            out_copy(k, k % _NBUF).wait()

    return sc_affine


def kernel(image, camindex, idindex, w, b):
    B, C, H, W = image.shape
    ws, bs = _gather_params(camindex, idindex, w, b)
    sc_affine = _make_sc_affine(B, C, H, W)
    return sc_affine(ws, bs, image)
